# baseline (speedup 1.0000x reference)
import functools

import jax
import jax.numpy as jnp
from jax import lax
from jax.experimental import pallas as pl
from jax.experimental.pallas import tpu as pltpu
from jax.experimental.pallas import tpu_sc as plsc

_N = 10000
_E = 320000
_D = 128
_H2 = 512
_LAT = 256
_NE = 8
_C = 40
_L = 4
_ORI = 0.5
_HALF = 128

_NSUB = 16
_CHUNK = 80
_EPW = _E // _NSUB
_NCHUNK = _EPW // _CHUNK
_SPLIT = 640
_LASTROWS = _N - 15 * _SPLIT
_CPT = _NCHUNK
_SUPER = 800
_SCH = _SUPER // _CHUNK
_NSUPER = _EPW // _SUPER
_EBUF = 3 * _SUPER
_RING = 3
_HIMASK = -65536


def _enc_body(x_ref, eps_ref, w1_ref, b1_ref, wmu_ref, bmu_ref, wlv_ref,
              blv_ref, zb_ref):
    x = x_ref[...]
    nrm = jnp.sqrt(jnp.sum(x * x, axis=1, keepdims=True))
    xn = x / jnp.maximum(nrm, 1e-12)
    h = lax.dot_general(xn, w1_ref[...], (((1,), (1,)), ((), ())),
                        preferred_element_type=jnp.float32) + b1_ref[...]
    h = jnp.maximum(h, 0.0)
    mu = lax.dot_general(h, wmu_ref[...], (((1,), (1,)), ((), ())),
                         preferred_element_type=jnp.float32) + bmu_ref[...]
    lv = lax.dot_general(h, wlv_ref[...], (((1,), (1,)), ((), ())),
                         preferred_element_type=jnp.float32) + blv_ref[...]
    z = mu + eps_ref[...] * jnp.exp(0.5 * lv)
    zn = jnp.sqrt(jnp.sum(z * z, axis=1, keepdims=True))
    z = z / jnp.maximum(zn, 1e-12)
    zb_ref[...] = z.astype(jnp.bfloat16)


def _encoder(x, eps, w1, b1, wmu, bmu, wlv, blv):
    bn = 1000
    grid = (_N // bn,)
    return pl.pallas_call(
        _enc_body,
        grid=grid,
        in_specs=[
            pl.BlockSpec((bn, _D), lambda i: (i, 0)),
            pl.BlockSpec((bn, _LAT), lambda i: (i, 0)),
            pl.BlockSpec((_H2, _D), lambda i: (0, 0)),
            pl.BlockSpec((1, _H2), lambda i: (0, 0)),
            pl.BlockSpec((_LAT, _H2), lambda i: (0, 0)),
            pl.BlockSpec((1, _LAT), lambda i: (0, 0)),
            pl.BlockSpec((_LAT, _H2), lambda i: (0, 0)),
            pl.BlockSpec((1, _LAT), lambda i: (0, 0)),
        ],
        out_specs=pl.BlockSpec((bn, _LAT), lambda i: (i, 0)),
        out_shape=jax.ShapeDtypeStruct((_N, _LAT), jnp.bfloat16),
    )(x, eps, w1, b1, wmu, bmu, wlv, blv)


def _spmm_body(src_hbm, dst_hbm, adj_hbm, z0, z1, out0, out1,
               acc, esrc, edst, eadj, gb0, gb1, fb0, fb1,
               gsem0, gsem1, ssem0, ssem1, esem):
    gbufs = (gb0, gb1)
    fbufs = (fb0, fb1)
    gsems = (gsem0, gsem1)
    ssems = (ssem0, ssem1)
    c = lax.axis_index("c")
    s = lax.axis_index("s")
    ebase = pl.multiple_of(s * _EPW, 8)
    rbase = pl.multiple_of(s * _SPLIT, 8)

    def _estage_sync(k):
        boff = pl.multiple_of(lax.rem(k, 3) * _SUPER, 8)
        hoff = pl.multiple_of(ebase + k * _SUPER, 8)
        pltpu.sync_copy(src_hbm.at[pl.ds(hoff, _SUPER)],
                        esrc.at[pl.ds(boff, _SUPER)])
        pltpu.sync_copy(dst_hbm.at[pl.ds(hoff, _SUPER)],
                        edst.at[pl.ds(boff, _SUPER)])
        pltpu.sync_copy(adj_hbm.at[pl.ds(hoff, _SUPER)],
                        eadj.at[pl.ds(boff, _SUPER)])

    def _estage(k):
        boff = pl.multiple_of(lax.rem(k, 3) * _SUPER, 8)
        hoff = pl.multiple_of(ebase + k * _SUPER, 8)
        pltpu.async_copy(src_hbm.at[pl.ds(hoff, _SUPER)],
                         esrc.at[pl.ds(boff, _SUPER)], esem)
        pltpu.async_copy(dst_hbm.at[pl.ds(hoff, _SUPER)],
                         edst.at[pl.ds(boff, _SUPER)], esem)
        pltpu.async_copy(adj_hbm.at[pl.ds(hoff, _SUPER)],
                         eadj.at[pl.ds(boff, _SUPER)], esem)

    def _ewait():
        for buf, hbm in ((esrc, src_hbm), (edst, dst_hbm), (eadj, adj_hbm)):
            pltpu.make_async_copy(hbm.at[pl.ds(ebase, _SUPER)],
                                  buf.at[pl.ds(0, _SUPER)], esem).wait()

    def _zero_fb0():
        def zb(r, carry):
            for j in range(_HALF // 16):
                fb0[r, pl.ds(j * 16, 16)] = jnp.zeros((16,), jnp.float32)
            return carry
        lax.fori_loop(0, _CHUNK, zb, 0)

    def _zero_acc(nrows):
        off = 0
        while off < nrows:
            step = min(_CHUNK, nrows - off)
            pltpu.sync_copy(fb0.at[pl.ds(0, step)],
                            acc.at[pl.ds(rbase + off, step)])
            off += step

    def _pack_out(out_c, tp1, nrows):
        off = 0
        while off < nrows:
            roff = pl.multiple_of(rbase + off, 16)
            pltpu.sync_copy(acc.at[pl.ds(roff, _CHUNK)], fb0)

            def packrow(r, carry):
                for j in range(_HALF // 32):
                    a = fb0[r, pl.ds(32 * j, 16)]
                    b = fb0[r, pl.ds(32 * j + 16, 16)]
                    ua = lax.bitcast_convert_type(a, jnp.int32)
                    ub = lax.bitcast_convert_type(b, jnp.int32)
                    ra = lax.shift_right_logical(
                        ua + 0x7FFF
                        + (lax.shift_right_logical(ua, 16) & 1), 16)
                    rb = (ub + 0x7FFF
                          + (lax.shift_right_logical(ub, 16) & 1)) & _HIMASK
                    gb0[r, pl.ds(16 * j, 16)] = ra | rb
                return carry
            lax.fori_loop(0, _CHUNK, packrow, 0)
            pltpu.sync_copy(gb0, out_c.at[tp1, pl.ds(roff, _CHUNK)])
            off += _CHUNK

    _zero_fb0()
    pl.when(s < _NSUB - 1)(functools.partial(_zero_acc, _SPLIT))
    pl.when(s == _NSUB - 1)(functools.partial(_zero_acc, _LASTROWS))
    plsc.subcore_barrier()

    def _chunks(table):
        def boff_of(ci):
            return pl.multiple_of(lax.rem(ci, 3 * _SCH) * _CHUNK, 8)

        def gsrc(ci):
            return table.at[esrc.at[pl.ds(boff_of(ci), _CHUNK)]]

        def sdst(ci):
            return acc.at[edst.at[pl.ds(boff_of(ci), _CHUNK)]]

        def gstart(ci, gb):
            sk = ci // _SCH

            @pl.when(lax.rem(ci, _SCH) == 0)
            def _():
                pl.when(ci > 0)(_ewait)
                pl.when(sk < _NSUPER - 1)(
                    functools.partial(_estage, sk + 1))
            pltpu.async_copy(gsrc(ci), gbufs[gb], gsems[gb])

        def gwait(ci, gb):
            pltpu.make_async_copy(gsrc(ci), gbufs[gb], gsems[gb]).wait()

        def sstart(ci, fb):
            pltpu.async_copy(fbufs[fb], sdst(ci), ssems[fb], add=True)

        def swait(ci, fb):
            pltpu.make_async_copy(fbufs[fb], sdst(ci), ssems[fb]).wait()

        def scale(ci, gb, fb):
            boff = boff_of(ci)
            gbuf = gbufs[gb]
            fbuf = fbufs[fb]

            def grp(g, carry2):
                wv = eadj[pl.ds(pl.multiple_of(boff + g * 16, 8), 16)]
                for k in range(16):
                    w = jnp.full((16,), wv[k], jnp.float32)
                    i = g * 16 + k
                    for j in range(_HALF // 32):
                        v = gbuf[i, pl.ds(16 * j, 16)]
                        a = lax.bitcast_convert_type(lax.shift_left(v, 16), jnp.float32)
                        b = lax.bitcast_convert_type(v & _HIMASK, jnp.float32)
                        fbuf[i, pl.ds(32 * j, 16)] = a * w
                        fbuf[i, pl.ds(32 * j + 16, 16)] = b * w
                return carry2
            lax.fori_loop(0, _CHUNK // 16, grp, 0)

        def stage(ci, par):
            gwait(ci, par)
            pl.when(ci >= 2)(functools.partial(swait, ci - 2, par))
            scale(ci, par, par)
            sstart(ci, par)
            pl.when(ci + 2 < _CPT)(functools.partial(gstart, ci + 2, par))

        _estage_sync(0)
        gstart(0, 0)
        gstart(1, 1)

        def pair(p, carry):
            stage(2 * p, 0)
            stage(2 * p + 1, 1)
            return carry
        lax.fori_loop(0, _CPT // 2, pair, 0)
        swait(_CPT - 2, 0)
        swait(_CPT - 1, 1)

    def _flush(out_c, tp1):
        def _own(nrows):
            _pack_out(out_c, tp1, nrows)
            _zero_fb0()
            _zero_acc(nrows)
        pl.when(s < _NSUB - 1)(functools.partial(_own, _SPLIT))
        pl.when(s == _NSUB - 1)(functools.partial(_own, _LASTROWS))

    def _zcopy(z_c, out_c):
        def _own(nrows):
            off = 0
            while off < nrows:
                roff = pl.multiple_of(rbase + off, 16)
                pltpu.sync_copy(z_c.at[pl.ds(roff, _CHUNK)], gb0)
                pltpu.sync_copy(gb0, out_c.at[0, pl.ds(roff, _CHUNK)])
                off += _CHUNK
        pl.when(s < _NSUB - 1)(functools.partial(_own, _SPLIT))
        pl.when(s == _NSUB - 1)(functools.partial(_own, _LASTROWS))

    pl.when(c == 0)(functools.partial(_zcopy, z0, out0))
    pl.when(c == 1)(functools.partial(_zcopy, z1, out1))
    plsc.subcore_barrier()

    def _hop(t, carry):
        pl.when(c == 0)(functools.partial(_chunks, out0.at[t]))
        pl.when(c == 1)(functools.partial(_chunks, out1.at[t]))
        plsc.subcore_barrier()
        pl.when(c == 0)(functools.partial(_flush, out0, t + 1))
        pl.when(c == 1)(functools.partial(_flush, out1, t + 1))
        plsc.subcore_barrier()
        return carry
    lax.fori_loop(0, _L, _hop, 0)


def _spmm(src, dst, adj, z0, z1):
    mesh = plsc.VectorSubcoreMesh(core_axis_name="c", subcore_axis_name="s")
    f = pl.kernel(
        _spmm_body,
        out_type=(
            jax.ShapeDtypeStruct((_L + 1, _N, _HALF // 2), jnp.int32),
            jax.ShapeDtypeStruct((_L + 1, _N, _HALF // 2), jnp.int32),
        ),
        mesh=mesh,
        compiler_params=pltpu.CompilerParams(use_tc_tiling_on_sc=False),
        scratch_types=[
            pltpu.VMEM_SHARED((_N, _HALF), jnp.float32),
            pltpu.VMEM((_EBUF,), jnp.int32),
            pltpu.VMEM((_EBUF,), jnp.int32),
            pltpu.VMEM((_EBUF,), jnp.float32),
            pltpu.VMEM((_CHUNK, _HALF // 2), jnp.int32),
            pltpu.VMEM((_CHUNK, _HALF // 2), jnp.int32),
            pltpu.VMEM((_CHUNK, _HALF), jnp.float32),
            pltpu.VMEM((_CHUNK, _HALF), jnp.float32),
            pltpu.SemaphoreType.DMA,
            pltpu.SemaphoreType.DMA,
            pltpu.SemaphoreType.DMA,
            pltpu.SemaphoreType.DMA,
            pltpu.SemaphoreType.DMA,
        ],
    )
    return f(src, dst, adj, z0, z1)


def _fuse_body(beta_ref, h0_ref, h1_ref, bias_ref, wg_ref,
               bg_ref, wef_ref, be_ref, o_ref):
    b = beta_ref[0, 0]
    f = jnp.tanh(b) + 1.0
    d = [jnp.float32(1.0), f, f * f, f * f * f]
    m = jnp.maximum(jnp.maximum(d[0], d[1]), jnp.maximum(d[2], d[3]))
    e = [jnp.exp(di - m) for di in d]
    tot = e[0] + e[1] + e[2] + e[3]
    w = [ei / tot for ei in e]

    def unpk(v):
        ev = lax.bitcast_convert_type(lax.shift_left(v, 16), jnp.float32)
        od = lax.bitcast_convert_type(v & _HIMASK, jnp.float32)
        parts = []
        for j in range(_HALF // 32):
            parts.append(ev[:, 16 * j:16 * j + 16])
            parts.append(od[:, 16 * j:16 * j + 16])
        return jnp.concatenate(parts, axis=1)

    h0 = h0_ref[...]
    h1 = h1_ref[...]
    f0 = (w[0] * unpk(h0[1]) + w[1] * unpk(h0[2])
          + w[2] * unpk(h0[3]) + w[3] * unpk(h0[4]))
    f1 = (w[0] * unpk(h1[1]) + w[1] * unpk(h1[2])
          + w[2] * unpk(h1[3]) + w[3] * unpk(h1[4]))
    fused = jnp.concatenate([f0, f1], axis=1) + jnp.tanh(bias_ref[...])
    hh = jnp.concatenate([unpk(h0[0]), unpk(h1[0])], axis=1)
    h2 = jnp.maximum(fused, 0.0) + _ORI * hh

    g = lax.dot_general(h2, wg_ref[...], (((1,), (1,)), ((), ())),
                        preferred_element_type=jnp.float32) + bg_ref[...]
    g = g - jnp.max(g, axis=1, keepdims=True)
    g = jnp.exp(g)
    g = g / jnp.sum(g, axis=1, keepdims=True)

    eo = lax.dot_general(h2, wef_ref[...], (((1,), (1,)), ((), ())),
                         preferred_element_type=jnp.float32)
    out = lax.dot_general(g, be_ref[...], (((1,), (0,)), ((), ())),
                          preferred_element_type=jnp.float32)
    for ei in range(_NE):
        out = out + g[:, ei:ei + 1] * eo[:, ei * _C:(ei + 1) * _C]

    mx = jnp.max(out, axis=1, keepdims=True)
    sh = out - mx
    lse = jnp.log(jnp.sum(jnp.exp(sh), axis=1, keepdims=True))
    o_ref[...] = sh - lse


def _fusion(beta, hops0, hops1, bias_p, wg, bg, wef, be):
    bn = 1000
    grid = (_N // bn,)
    return pl.pallas_call(
        _fuse_body,
        grid=grid,
        in_specs=[
            pl.BlockSpec((1, 1), lambda i: (0, 0)),
            pl.BlockSpec((_L + 1, bn, _HALF // 2), lambda i: (0, i, 0)),
            pl.BlockSpec((_L + 1, bn, _HALF // 2), lambda i: (0, i, 0)),
            pl.BlockSpec((bn, _LAT), lambda i: (i, 0)),
            pl.BlockSpec((_NE, _LAT), lambda i: (0, 0)),
            pl.BlockSpec((1, _NE), lambda i: (0, 0)),
            pl.BlockSpec((_NE * _C, _LAT), lambda i: (0, 0)),
            pl.BlockSpec((_NE, _C), lambda i: (0, 0)),
        ],
        out_specs=pl.BlockSpec((bn, _C), lambda i: (i, 0)),
        out_shape=jax.ShapeDtypeStruct((_N, _C), jnp.float32),
    )(beta, hops0, hops1, bias_p, wg, bg, wef, be)


def kernel(x, edge_index, adj_w, eps, W1, b1, Wmu, bmu, Wlv, blv, Wg, bg,
           We, be, beta, bias_p):
    src = edge_index[0]
    dst = edge_index[1]
    zb = _encoder(x, eps, W1, jnp.reshape(b1, (1, _H2)),
                  Wmu, jnp.reshape(bmu, (1, _LAT)),
                  Wlv, jnp.reshape(blv, (1, _LAT)))
    zi = lax.bitcast_convert_type(jnp.reshape(zb, (_N, _HALF, 2)), jnp.int32)
    z0i = zi[:, :_HALF // 2]
    z1i = zi[:, _HALF // 2:]
    hops0, hops1 = _spmm(src, dst, adj_w, z0i, z1i)

    def perm(a):
        sh = a.shape[:-1]
        return jnp.reshape(
            jnp.swapaxes(jnp.reshape(a, sh + (_LAT // 32, 16, 2)), -1, -2),
            sh + (_LAT,))

    beta2 = jnp.reshape(jnp.asarray(beta, jnp.float32), (1, 1))
    wef = jnp.reshape(We, (_NE * _C, _LAT))
    return _fusion(beta2, hops0, hops1, perm(bias_p),
                   perm(Wg), jnp.reshape(bg, (1, _NE)), perm(wef), be)


# speedup vs baseline: 2.3089x; 2.3089x over previous
import functools

import jax
import jax.numpy as jnp
from jax import lax
from jax.experimental import pallas as pl
from jax.experimental.pallas import tpu as pltpu
from jax.experimental.pallas import tpu_sc as plsc

_N = 10000
_E = 320000
_D = 128
_H2 = 512
_LAT = 256
_NE = 8
_C = 40
_L = 4
_ORI = 0.5
_HALF = 128

_NSUB = 16
_CHUNK = 80
_EPW = _E // _NSUB
_NCHUNK = _EPW // _CHUNK
_SPLIT = 632
_LASTROWS = _N - 15 * _SPLIT
_CPT = _NCHUNK
_SUPER = 800
_SCH = _SUPER // _CHUNK
_NSUPER = _EPW // _SUPER
_EBUF = 3 * _SUPER
_RING = 3


def _enc_body(x_ref, eps_ref, w1_ref, b1_ref, wmu_ref, bmu_ref, wlv_ref,
              blv_ref, z0_ref, z1_ref):
    x = x_ref[...]
    nrm = jnp.sqrt(jnp.sum(x * x, axis=1, keepdims=True))
    xn = x / jnp.maximum(nrm, 1e-12)
    h = lax.dot_general(xn, w1_ref[...], (((1,), (1,)), ((), ())),
                        preferred_element_type=jnp.float32) + b1_ref[...]
    h = jnp.maximum(h, 0.0)
    mu = lax.dot_general(h, wmu_ref[...], (((1,), (1,)), ((), ())),
                         preferred_element_type=jnp.float32) + bmu_ref[...]
    lv = lax.dot_general(h, wlv_ref[...], (((1,), (1,)), ((), ())),
                         preferred_element_type=jnp.float32) + blv_ref[...]
    z = mu + eps_ref[...] * jnp.exp(0.5 * lv)
    zn = jnp.sqrt(jnp.sum(z * z, axis=1, keepdims=True))
    z = z / jnp.maximum(zn, 1e-12)
    z0_ref[...] = z[:, :_HALF]
    z1_ref[...] = z[:, _HALF:]


def _encoder(x, eps, w1, b1, wmu, bmu, wlv, blv):
    bn = 2000
    grid = (_N // bn,)
    return pl.pallas_call(
        _enc_body,
        grid=grid,
        in_specs=[
            pl.BlockSpec((bn, _D), lambda i: (i, 0)),
            pl.BlockSpec((bn, _LAT), lambda i: (i, 0)),
            pl.BlockSpec((_H2, _D), lambda i: (0, 0)),
            pl.BlockSpec((1, _H2), lambda i: (0, 0)),
            pl.BlockSpec((_LAT, _H2), lambda i: (0, 0)),
            pl.BlockSpec((1, _LAT), lambda i: (0, 0)),
            pl.BlockSpec((_LAT, _H2), lambda i: (0, 0)),
            pl.BlockSpec((1, _LAT), lambda i: (0, 0)),
        ],
        out_specs=[
            pl.BlockSpec((bn, _HALF), lambda i: (i, 0)),
            pl.BlockSpec((bn, _HALF), lambda i: (i, 0)),
        ],
        out_shape=[
            jax.ShapeDtypeStruct((_N, _HALF), jnp.float32),
            jax.ShapeDtypeStruct((_N, _HALF), jnp.float32),
        ],
    )(x, eps, w1, b1, wmu, bmu, wlv, blv)


def _spmm_body(src_hbm, dst_hbm, adj_hbm, z0, z1, out0, out1,
               acc, esrc, edst, eadj, rows0, rows1, rows2,
               gsem0, gsem1, gsem2, ssem0, ssem1, ssem2, esem):
    bufs = (rows0, rows1, rows2)
    gsems = (gsem0, gsem1, gsem2)
    ssems = (ssem0, ssem1, ssem2)
    c = lax.axis_index("c")
    s = lax.axis_index("s")
    ebase = pl.multiple_of(s * _EPW, 8)
    rbase = pl.multiple_of(s * _SPLIT, 8)

    def _estage_sync(k):
        boff = pl.multiple_of(lax.rem(k, 3) * _SUPER, 8)
        hoff = pl.multiple_of(ebase + k * _SUPER, 8)
        pltpu.sync_copy(src_hbm.at[pl.ds(hoff, _SUPER)],
                        esrc.at[pl.ds(boff, _SUPER)])
        pltpu.sync_copy(dst_hbm.at[pl.ds(hoff, _SUPER)],
                        edst.at[pl.ds(boff, _SUPER)])
        pltpu.sync_copy(adj_hbm.at[pl.ds(hoff, _SUPER)],
                        eadj.at[pl.ds(boff, _SUPER)])

    def _estage(k):
        boff = pl.multiple_of(lax.rem(k, 3) * _SUPER, 8)
        hoff = pl.multiple_of(ebase + k * _SUPER, 8)
        pltpu.async_copy(src_hbm.at[pl.ds(hoff, _SUPER)],
                         esrc.at[pl.ds(boff, _SUPER)], esem)
        pltpu.async_copy(dst_hbm.at[pl.ds(hoff, _SUPER)],
                         edst.at[pl.ds(boff, _SUPER)], esem)
        pltpu.async_copy(adj_hbm.at[pl.ds(hoff, _SUPER)],
                         eadj.at[pl.ds(boff, _SUPER)], esem)

    def _ewait():
        for buf, hbm in ((esrc, src_hbm), (edst, dst_hbm), (eadj, adj_hbm)):
            pltpu.make_async_copy(hbm.at[pl.ds(ebase, _SUPER)],
                                  buf.at[pl.ds(0, _SUPER)], esem).wait()

    def _zero_rows0():
        def zb(r, carry):
            for j in range(_HALF // 16):
                rows0[r, pl.ds(j * 16, 16)] = jnp.zeros((16,), jnp.float32)
            return carry
        lax.fori_loop(0, _CHUNK, zb, 0)

    def _zero_acc(nrows):
        off = 0
        while off < nrows:
            step = min(_CHUNK, nrows - off)
            pltpu.sync_copy(rows0.at[pl.ds(0, step)],
                            acc.at[pl.ds(rbase + off, step)])
            off += step

    def _copy_out(out_t, nrows):
        off = 0
        while off < nrows:
            step = min(_CHUNK, nrows - off)
            pltpu.sync_copy(acc.at[pl.ds(rbase + off, step)],
                            out_t.at[pl.ds(rbase + off, step)])
            off += step

    _zero_rows0()
    pl.when(s < _NSUB - 1)(functools.partial(_zero_acc, _SPLIT))
    pl.when(s == _NSUB - 1)(functools.partial(_zero_acc, _LASTROWS))
    plsc.subcore_barrier()

    def _chunks(table):
        def boff_of(ci):
            return pl.multiple_of(lax.rem(ci, 3 * _SCH) * _CHUNK, 8)

        def gsrc(ci):
            return table.at[esrc.at[pl.ds(boff_of(ci), _CHUNK)]]

        def sdst(ci):
            return acc.at[edst.at[pl.ds(boff_of(ci), _CHUNK)]]

        def gstart(ci, b):
            sk = ci // _SCH

            @pl.when(lax.rem(ci, _SCH) == 0)
            def _():
                pl.when(ci > 0)(_ewait)
                pl.when(sk < _NSUPER - 1)(
                    functools.partial(_estage, sk + 1))
            pltpu.async_copy(gsrc(ci), bufs[b], gsems[b])

        def gwait(ci, b):
            pltpu.make_async_copy(gsrc(ci), bufs[b], gsems[b]).wait()

        def sstart(ci, b):
            pltpu.async_copy(bufs[b], sdst(ci), ssems[b], add=True)

        def swait(ci, b):
            pltpu.make_async_copy(bufs[b], sdst(ci), ssems[b]).wait()

        def scale(ci, b):
            boff = boff_of(ci)
            buf = bufs[b]

            def grp(g, carry2):
                wv = eadj[pl.ds(pl.multiple_of(boff + g * 16, 8), 16)]
                for k in range(16):
                    w = jnp.full((16,), wv[k], jnp.float32)
                    i = g * 16 + k
                    for j in range(_HALF // 16):
                        buf[i, pl.ds(j * 16, 16)] = (
                            buf[i, pl.ds(j * 16, 16)] * w)
                return carry2
            lax.fori_loop(0, _CHUNK // 16, grp, 0)

        def stage(ci, b):
            gwait(ci, b)
            scale(ci, b)
            sstart(ci, b)
            pl.when(ci >= 1)(functools.partial(swait, ci - 1,
                                               (b + _RING - 1) % _RING))
            pl.when(ci + 2 < _CPT)(functools.partial(gstart, ci + 2,
                                                     (b + 2) % _RING))

        _estage_sync(0)
        gstart(0, 0)
        gstart(1, 1)

        def triple(q, carry):
            c0 = 3 * q
            stage(c0, 0)
            stage(c0 + 1, 1)
            stage(c0 + 2, 2)
            return carry
        lax.fori_loop(0, _CPT // 3, triple, 0)
        stage(_CPT - 1, (_CPT - 1) % _RING)
        swait(_CPT - 1, (_CPT - 1) % _RING)
        _zero_rows0()

    def _flush(out_t):
        def _own(nrows):
            _copy_out(out_t, nrows)
            _zero_acc(nrows)
        pl.when(s < _NSUB - 1)(functools.partial(_own, _SPLIT))
        pl.when(s == _NSUB - 1)(functools.partial(_own, _LASTROWS))

    for t in range(_L):
        t0 = z0 if t == 0 else out0.at[t - 1]
        t1 = z1 if t == 0 else out1.at[t - 1]
        pl.when(c == 0)(functools.partial(_chunks, t0))
        pl.when(c == 1)(functools.partial(_chunks, t1))
        plsc.subcore_barrier()
        pl.when(c == 0)(functools.partial(_flush, out0.at[t]))
        pl.when(c == 1)(functools.partial(_flush, out1.at[t]))
        plsc.subcore_barrier()


def _spmm(src, dst, adj, z0, z1):
    mesh = plsc.VectorSubcoreMesh(core_axis_name="c", subcore_axis_name="s")
    f = pl.kernel(
        _spmm_body,
        out_type=(
            jax.ShapeDtypeStruct((_L, _N, _HALF), jnp.float32),
            jax.ShapeDtypeStruct((_L, _N, _HALF), jnp.float32),
        ),
        mesh=mesh,
        scratch_types=[
            pltpu.VMEM_SHARED((_N, _HALF), jnp.float32),
            pltpu.VMEM((_EBUF,), jnp.int32),
            pltpu.VMEM((_EBUF,), jnp.int32),
            pltpu.VMEM((_EBUF,), jnp.float32),
            pltpu.VMEM((_CHUNK, _HALF), jnp.float32),
            pltpu.VMEM((_CHUNK, _HALF), jnp.float32),
            pltpu.VMEM((_CHUNK, _HALF), jnp.float32),
            pltpu.SemaphoreType.DMA,
            pltpu.SemaphoreType.DMA,
            pltpu.SemaphoreType.DMA,
            pltpu.SemaphoreType.DMA,
            pltpu.SemaphoreType.DMA,
            pltpu.SemaphoreType.DMA,
            pltpu.SemaphoreType.DMA,
        ],
    )
    return f(src, dst, adj, z0, z1)


def _fuse_body(beta_ref, h0_ref, h1_ref, z0_ref, z1_ref, bias_ref, wg_ref,
               bg_ref, wef_ref, be_ref, o_ref):
    b = beta_ref[0, 0]
    f = jnp.tanh(b) + 1.0
    d = [jnp.float32(1.0), f, f * f, f * f * f]
    m = jnp.maximum(jnp.maximum(d[0], d[1]), jnp.maximum(d[2], d[3]))
    e = [jnp.exp(di - m) for di in d]
    tot = e[0] + e[1] + e[2] + e[3]
    w = [ei / tot for ei in e]

    h0 = h0_ref[...]
    h1 = h1_ref[...]
    f0 = w[0] * h0[0] + w[1] * h0[1] + w[2] * h0[2] + w[3] * h0[3]
    f1 = w[0] * h1[0] + w[1] * h1[1] + w[2] * h1[2] + w[3] * h1[3]
    fused = jnp.concatenate([f0, f1], axis=1) + jnp.tanh(bias_ref[...])
    hh = jnp.concatenate([z0_ref[...], z1_ref[...]], axis=1)
    h2 = jnp.maximum(fused, 0.0) + _ORI * hh

    g = lax.dot_general(h2, wg_ref[...], (((1,), (1,)), ((), ())),
                        preferred_element_type=jnp.float32) + bg_ref[...]
    g = g - jnp.max(g, axis=1, keepdims=True)
    g = jnp.exp(g)
    g = g / jnp.sum(g, axis=1, keepdims=True)

    eo = lax.dot_general(h2, wef_ref[...], (((1,), (1,)), ((), ())),
                         preferred_element_type=jnp.float32)
    out = lax.dot_general(g, be_ref[...], (((1,), (0,)), ((), ())),
                          preferred_element_type=jnp.float32)
    for ei in range(_NE):
        out = out + g[:, ei:ei + 1] * eo[:, ei * _C:(ei + 1) * _C]

    mx = jnp.max(out, axis=1, keepdims=True)
    sh = out - mx
    lse = jnp.log(jnp.sum(jnp.exp(sh), axis=1, keepdims=True))
    o_ref[...] = sh - lse


def _fusion(beta, hops0, hops1, z0, z1, bias_p, wg, bg, wef, be):
    bn = 2000
    grid = (_N // bn,)
    return pl.pallas_call(
        _fuse_body,
        grid=grid,
        in_specs=[
            pl.BlockSpec((1, 1), lambda i: (0, 0)),
            pl.BlockSpec((_L, bn, _HALF), lambda i: (0, i, 0)),
            pl.BlockSpec((_L, bn, _HALF), lambda i: (0, i, 0)),
            pl.BlockSpec((bn, _HALF), lambda i: (i, 0)),
            pl.BlockSpec((bn, _HALF), lambda i: (i, 0)),
            pl.BlockSpec((bn, _LAT), lambda i: (i, 0)),
            pl.BlockSpec((_NE, _LAT), lambda i: (0, 0)),
            pl.BlockSpec((1, _NE), lambda i: (0, 0)),
            pl.BlockSpec((_NE * _C, _LAT), lambda i: (0, 0)),
            pl.BlockSpec((_NE, _C), lambda i: (0, 0)),
        ],
        out_specs=pl.BlockSpec((bn, _C), lambda i: (i, 0)),
        out_shape=jax.ShapeDtypeStruct((_N, _C), jnp.float32),
    )(beta, hops0, hops1, z0, z1, bias_p, wg, bg, wef, be)


def kernel(x, edge_index, adj_w, eps, W1, b1, Wmu, bmu, Wlv, blv, Wg, bg,
           We, be, beta, bias_p):
    src = edge_index[0]
    dst = edge_index[1]
    z0, z1 = _encoder(x, eps, W1, jnp.reshape(b1, (1, _H2)),
                      Wmu, jnp.reshape(bmu, (1, _LAT)),
                      Wlv, jnp.reshape(blv, (1, _LAT)))
    hops0, hops1 = _spmm(src, dst, adj_w, z0, z1)
    beta2 = jnp.reshape(jnp.asarray(beta, jnp.float32), (1, 1))
    wef = jnp.reshape(We, (_NE * _C, _LAT))
    return _fusion(beta2, hops0, hops1, z0, z1, bias_p,
                   Wg, jnp.reshape(bg, (1, _NE)), wef, be)
